# Initial kernel scaffold; baseline (speedup 1.0000x reference)
import jax
import jax.numpy as jnp
from jax.experimental import pallas as pl


def kernel(data_list, graph, fc_w0, fc_b0, fc1_w0, fc1_b0, fc_w1, fc_b1, fc1_w1, fc1_b1, fc_w2, fc_b2, fc1_w2, fc1_b2, gcn_w, gcn_b, cls_w0, cls_b0, cls_w1, cls_b1):
    raise NotImplementedError("write your pallas kernel here")


# speedup vs baseline: 1052.7032x; 1052.7032x over previous
import jax
import jax.numpy as jnp
from jax.experimental import pallas as pl


def _dot(a, b):
    return jax.lax.dot_general(
        a, b, (((1,), (0,)), ((), ())), preferred_element_type=jnp.float32
    )


def _gnn_fwd(
    data_ref, graph_ref,
    fw0, fb0, f1w0, f1b0,
    fw1, fb1, f1w1, f1b1,
    fw2, fb2, f1w2, f1b2,
    gw, gb, cw0, cb0, cw1, cb1,
    out_ref,
):
    feats = []
    for i, (fw, fb, f1w, f1b) in enumerate(
        ((fw0, fb0, f1w0, f1b0), (fw1, fb1, f1w1, f1b1), (fw2, fb2, f1w2, f1b2))
    ):
        x = data_ref[i]
        h = jnp.maximum(_dot(x, fw[...]) + fb[...], 0.0)
        h = jnp.maximum(_dot(h, f1w[...]) + f1b[...], 0.0)
        feats.append(h)
    mm = jnp.concatenate(feats, axis=1)

    xw = _dot(mm, gw[...])
    graph = graph_ref[...]
    deg = jnp.sum(graph, axis=0) + 1.0
    dis = jnp.where(deg > 0, jax.lax.rsqrt(jnp.maximum(deg, 1e-12)), 0.0)
    sx = xw * dis[:, None]
    y = jax.lax.dot_general(
        graph, sx, (((0,), (0,)), ((), ())), preferred_element_type=jnp.float32
    )
    g = dis[:, None] * (y + sx) + gb[...]

    z = jnp.concatenate([mm, g], axis=1)
    h = _dot(z, cw0[...]) + cb0[...]
    h = jnp.where(h >= 0, h, 0.01 * h)
    out_ref[...] = _dot(h, cw1[...]) + cb1[...]


def kernel(data_list, graph, fc_w0, fc_b0, fc1_w0, fc1_b0, fc_w1, fc_b1,
           fc1_w1, fc1_b1, fc_w2, fc_b2, fc1_w2, fc1_b2, gcn_w, gcn_b,
           cls_w0, cls_b0, cls_w1, cls_b1):
    N = graph.shape[0]
    C = cls_w1.shape[0]
    args = (
        data_list, graph,
        fc_w0.T, fc_b0[None, :], fc1_w0.T, fc1_b0[None, :],
        fc_w1.T, fc_b1[None, :], fc1_w1.T, fc1_b1[None, :],
        fc_w2.T, fc_b2[None, :], fc1_w2.T, fc1_b2[None, :],
        gcn_w.T, gcn_b[None, :],
        cls_w0.T, cls_b0[None, :],
        cls_w1.T, cls_b1[None, :],
    )
    return pl.pallas_call(
        _gnn_fwd,
        out_shape=jax.ShapeDtypeStruct((N, C), jnp.float32),
    )(*args)


# speedup vs baseline: 2503.0842x; 2.3778x over previous
import jax
import jax.numpy as jnp
from jax.experimental import pallas as pl


def _dot_nt(a, b):
    return jax.lax.dot_general(
        a, b, (((1,), (1,)), ((), ())), preferred_element_type=jnp.float32
    )


def _gnn_fwd(
    data_ref, graph_ref,
    fw0, fb0, f1w0, f1b0,
    fw1, fb1, f1w1, f1b1,
    fw2, fb2, f1w2, f1b2,
    gw, gb, cw0, cb0, cw1, cb1,
    out_ref,
):
    feats = []
    for i, (fw, fb, f1w, f1b) in enumerate(
        ((fw0, fb0, f1w0, f1b0), (fw1, fb1, f1w1, f1b1), (fw2, fb2, f1w2, f1b2))
    ):
        x = data_ref[i]
        h = jnp.maximum(_dot_nt(x, fw[...]) + fb[...], 0.0)
        h = jnp.maximum(_dot_nt(h, f1w[...]) + f1b[...], 0.0)
        feats.append(h)
    mm = jnp.concatenate(feats, axis=1)

    xw = _dot_nt(mm, gw[...])
    graph = graph_ref[...]
    deg = jnp.sum(graph, axis=0) + 1.0
    dis = jnp.where(deg > 0, jax.lax.rsqrt(jnp.maximum(deg, 1e-12)), 0.0)
    sx = xw * dis[:, None]
    y = jax.lax.dot_general(
        graph, sx, (((0,), (0,)), ((), ())), preferred_element_type=jnp.float32
    )
    g = dis[:, None] * (y + sx) + gb[...]

    z = jnp.concatenate([mm, g], axis=1)
    h = _dot_nt(z, cw0[...]) + cb0[...]
    h = jnp.where(h >= 0, h, 0.01 * h)
    out_ref[...] = _dot_nt(h, cw1[...]) + cb1[...]


def kernel(data_list, graph, fc_w0, fc_b0, fc1_w0, fc1_b0, fc_w1, fc_b1,
           fc1_w1, fc1_b1, fc_w2, fc_b2, fc1_w2, fc1_b2, gcn_w, gcn_b,
           cls_w0, cls_b0, cls_w1, cls_b1):
    N = graph.shape[0]
    C = cls_w1.shape[0]
    args = (
        data_list, graph,
        fc_w0, fc_b0, fc1_w0, fc1_b0,
        fc_w1, fc_b1, fc1_w1, fc1_b1,
        fc_w2, fc_b2, fc1_w2, fc1_b2,
        gcn_w, gcn_b,
        cls_w0, cls_b0,
        cls_w1, cls_b1,
    )
    return pl.pallas_call(
        _gnn_fwd,
        out_shape=jax.ShapeDtypeStruct((N, C), jnp.float32),
    )(*args)
